# baseline (speedup 1.0000x reference)
import functools

import jax
import jax.numpy as jnp
from jax import lax
from jax.experimental import pallas as pl
from jax.experimental.pallas import tpu as pltpu
from jax.experimental.pallas import tpu_sc as plsc

N_NODES = 10000
N_EDGES = 320000
NC = 2
NS = 16
NW = NC * NS
CHUNK = 128
N_CHUNKS = N_EDGES // CHUNK
ITERS = -(-N_CHUNKS // NW)
ROWS_PER_TILE = (N_NODES // NS) // 8 * 8
TAIL_BASE = ROWS_PER_TILE * NS
TAIL = N_NODES - TAIL_BASE
CPT = 80
E_PAD = NW * CPT * CHUNK
AGG_ROWS = N_NODES + 128
NBUF = 2
PHASES = 2
CPH = CPT // PHASES

ROW_BLK = 1000
GRID = N_NODES // ROW_BLK


def _seg_sum_partials(support, src, dst, zeros, d):
    mesh = plsc.VectorSubcoreMesh(
        core_axis_name="c", subcore_axis_name="s", num_cores=NC, num_subcores=NS
    )

    @functools.partial(
        pl.kernel,
        out_type=jax.ShapeDtypeStruct((NC, N_NODES, d), jnp.float32),
        mesh=mesh,
        scratch_types=[
            pltpu.VMEM((CPH, CHUNK), jnp.int32),
            pltpu.VMEM((CPH, CHUNK), jnp.int32),
            [pltpu.VMEM((CHUNK, d), jnp.float32) for _ in range(NBUF)],
            pltpu.VMEM_SHARED((AGG_ROWS, d), jnp.float32),
            [pltpu.SemaphoreType.DMA for _ in range(NBUF)],
            [pltpu.SemaphoreType.DMA for _ in range(NBUF)],
        ],
    )
    def k(support_hbm, src_hbm, dst_hbm, zeros_hbm, out_hbm,
          src_idx, dst_idx, rows, agg_sh, gsem, ssem):
        cid = lax.axis_index("c")
        sid = lax.axis_index("s")
        wid = sid * NC + cid

        pltpu.sync_copy(zeros_hbm, agg_sh.at[pl.ds(sid * ROWS_PER_TILE, ROWS_PER_TILE)])

        @pl.when(sid == 0)
        def _():
            pltpu.sync_copy(
                zeros_hbm.at[pl.ds(0, AGG_ROWS - TAIL_BASE)],
                agg_sh.at[pl.ds(TAIL_BASE, AGG_ROWS - TAIL_BASE)],
            )

        plsc.subcore_barrier()

        def gather_start(i, b):
            pltpu.async_copy(support_hbm.at[src_idx.at[i]], rows[b], gsem[b])

        def gather_wait(i, b):
            pltpu.make_async_copy(support_hbm.at[src_idx.at[i]], rows[b], gsem[b]).wait()

        def scatter_start(i, b):
            pltpu.async_copy(rows[b], agg_sh.at[dst_idx.at[i]], ssem[b], add=True)

        def scatter_wait(i, b):
            pltpu.make_async_copy(rows[b], agg_sh.at[dst_idx.at[i]], ssem[b]).wait()

        for ph in range(PHASES):
            base_row = wid * CPT + ph * CPH
            pltpu.sync_copy(src_hbm.at[pl.ds(base_row, CPH)], src_idx)
            pltpu.sync_copy(dst_hbm.at[pl.ds(base_row, CPH)], dst_idx)

            def body(i, carry):
                pltpu.sync_copy(support_hbm.at[src_idx.at[i]], rows[0])
                pltpu.sync_copy(rows[0], agg_sh.at[dst_idx.at[i]], add=True)
                return carry

            lax.fori_loop(0, CPH, body, None)

        plsc.subcore_barrier()
        pltpu.sync_copy(
            agg_sh.at[pl.ds(sid * ROWS_PER_TILE, ROWS_PER_TILE)],
            out_hbm.at[cid, pl.ds(sid * ROWS_PER_TILE, ROWS_PER_TILE)],
        )

        @pl.when(sid == 0)
        def _():
            pltpu.sync_copy(
                agg_sh.at[pl.ds(TAIL_BASE, TAIL)],
                out_hbm.at[cid, pl.ds(TAIL_BASE, TAIL)],
            )

    return k(support, src, dst, zeros)


def _mm_bias_relu(parts, w, b, n_out):
    def body(p_ref, w_ref, b_ref, o_ref):
        agg = p_ref[0] + p_ref[1]
        o_ref[...] = jnp.maximum(
            jnp.dot(agg, w_ref[...], preferred_element_type=jnp.float32) + b_ref[...],
            0.0,
        )

    d = parts.shape[2]
    return pl.pallas_call(
        body,
        grid=(GRID,),
        in_specs=[
            pl.BlockSpec((NC, ROW_BLK, d), lambda i: (0, i, 0)),
            pl.BlockSpec(w.shape, lambda i: (0, 0)),
            pl.BlockSpec((1, n_out), lambda i: (0, 0)),
        ],
        out_specs=pl.BlockSpec((ROW_BLK, n_out), lambda i: (i, 0)),
        out_shape=jax.ShapeDtypeStruct((N_NODES, n_out), jnp.float32),
    )(parts, w, b.reshape(1, n_out))


def _mm_bias_log_softmax(parts, w, b, n_out):
    def body(p_ref, w_ref, b_ref, o_ref):
        agg = p_ref[0] + p_ref[1]
        o = jnp.dot(agg, w_ref[...], preferred_element_type=jnp.float32) + b_ref[...]
        m = jnp.max(o, axis=1, keepdims=True)
        e = jnp.exp(o - m)
        s = jnp.sum(e, axis=1, keepdims=True)
        o_ref[...] = o - m - jnp.log(s)

    d = parts.shape[2]
    return pl.pallas_call(
        body,
        grid=(GRID,),
        in_specs=[
            pl.BlockSpec((NC, ROW_BLK, d), lambda i: (0, i, 0)),
            pl.BlockSpec(w.shape, lambda i: (0, 0)),
            pl.BlockSpec((1, n_out), lambda i: (0, 0)),
        ],
        out_specs=pl.BlockSpec((ROW_BLK, n_out), lambda i: (i, 0)),
        out_shape=jax.ShapeDtypeStruct((N_NODES, n_out), jnp.float32),
    )(parts, w, b.reshape(1, n_out))


def kernel(x, adjs, W0, b0, W1, b1):
    pad = E_PAD - N_EDGES
    src = jnp.concatenate(
        [adjs[0].astype(jnp.int32), jnp.zeros((pad,), jnp.int32)]
    ).reshape(NW * CPT, CHUNK)
    dst = jnp.concatenate(
        [adjs[1].astype(jnp.int32),
         N_NODES + (jnp.arange(pad, dtype=jnp.int32) % 128)]
    ).reshape(NW * CPT, CHUNK)
    nfeat = x.shape[1]
    nhid = W0.shape[1]
    ncls = W1.shape[1]
    z = jnp.zeros((ROWS_PER_TILE, nfeat), jnp.float32)

    parts1 = _seg_sum_partials(x, src, dst, z, nfeat)
    h = _mm_bias_relu(parts1, W0, b0, nhid)
    parts2 = _seg_sum_partials(h, src, dst, z, nhid)
    return _mm_bias_log_softmax(parts2, W1, b1, ncls)


# speedup vs baseline: 1.0066x; 1.0066x over previous
import functools

import jax
import jax.numpy as jnp
from jax import lax
from jax.experimental import pallas as pl
from jax.experimental.pallas import tpu as pltpu
from jax.experimental.pallas import tpu_sc as plsc

N_NODES = 10000
N_EDGES = 320000
NC = 2
NS = 16
NW = NC * NS
CHUNK = 128
N_CHUNKS = N_EDGES // CHUNK
ITERS = -(-N_CHUNKS // NW)
ROWS_PER_TILE = (N_NODES // NS) // 8 * 8
TAIL_BASE = ROWS_PER_TILE * NS
TAIL = N_NODES - TAIL_BASE
CPT = 80
E_PAD = NW * CPT * CHUNK
AGG_ROWS = N_NODES + 16
NBUF = 3
NIDX = 6

ROW_BLK = 1000
GRID = N_NODES // ROW_BLK


def _seg_sum_partials(support, src, dst, zeros, d):
    mesh = plsc.VectorSubcoreMesh(
        core_axis_name="c", subcore_axis_name="s", num_cores=NC, num_subcores=NS
    )

    @functools.partial(
        pl.kernel,
        out_type=jax.ShapeDtypeStruct((NC, N_NODES, d), jnp.float32),
        mesh=mesh,
        scratch_types=[
            [pltpu.VMEM((CHUNK,), jnp.int32) for _ in range(NIDX)],
            [pltpu.VMEM((CHUNK,), jnp.int32) for _ in range(NIDX)],
            [pltpu.VMEM((CHUNK, d), jnp.float32) for _ in range(NBUF)],
            pltpu.VMEM_SHARED((AGG_ROWS, d), jnp.float32),
            [pltpu.SemaphoreType.DMA for _ in range(NIDX)],
            [pltpu.SemaphoreType.DMA for _ in range(NBUF)],
            [pltpu.SemaphoreType.DMA for _ in range(NBUF)],
        ],
    )
    def k(support_hbm, src_hbm, dst_hbm, zeros_hbm, out_hbm,
          src_idx, dst_idx, rows, agg_sh, isem, gsem, ssem):
        cid = lax.axis_index("c")
        sid = lax.axis_index("s")
        wid = sid * NC + cid

        pltpu.sync_copy(zeros_hbm, agg_sh.at[pl.ds(sid * ROWS_PER_TILE, ROWS_PER_TILE)])

        @pl.when(sid == 0)
        def _():
            pltpu.sync_copy(
                zeros_hbm.at[pl.ds(0, AGG_ROWS - TAIL_BASE)],
                agg_sh.at[pl.ds(TAIL_BASE, AGG_ROWS - TAIL_BASE)],
            )

        plsc.subcore_barrier()

        def gather_start(i, b):
            pltpu.async_copy(support_hbm.at[src_idx.at[i]], rows[b], gsem[b])

        def gather_wait(i, b):
            pltpu.make_async_copy(support_hbm.at[src_idx.at[i]], rows[b], gsem[b]).wait()

        def scatter_start(i, b):
            pltpu.async_copy(rows[b], agg_sh.at[dst_idx.at[i]], ssem[b], add=True)

        def scatter_wait(i, b):
            pltpu.make_async_copy(rows[b], agg_sh.at[dst_idx.at[i]], ssem[b]).wait()

        def idx_start(i, bi):
            base = (wid * CPT + i) * CHUNK
            pltpu.async_copy(src_hbm.at[pl.ds(base, CHUNK)], src_idx[bi], isem[bi])
            pltpu.async_copy(dst_hbm.at[pl.ds(base, CHUNK)], dst_idx[bi], isem[bi])

        def idx_wait(bi):
            pltpu.make_async_copy(src_hbm.at[pl.ds(0, CHUNK)], src_idx[bi], isem[bi]).wait()
            pltpu.make_async_copy(dst_hbm.at[pl.ds(0, CHUNK)], dst_idx[bi], isem[bi]).wait()

        def gather_start(br, bi):
            pltpu.async_copy(support_hbm.at[src_idx[bi]], rows[br], gsem[br])

        def gather_wait(br, bi):
            pltpu.make_async_copy(support_hbm.at[src_idx[bi]], rows[br], gsem[br]).wait()

        def scatter_start(br, bi):
            pltpu.async_copy(rows[br], agg_sh.at[dst_idx[bi]], ssem[br], add=True)

        def scatter_wait(br, bi):
            pltpu.make_async_copy(rows[br], agg_sh.at[dst_idx[bi]], ssem[br]).wait()

        def step(i, ph, with_g=True, with_i=True, with_sw=True):
            br, bi = ph % NBUF, ph % NIDX
            gather_wait(br, bi)
            scatter_start(br, bi)
            if with_g:
                brn, bin_ = (ph + 1) % NBUF, (ph + 1) % NIDX
                idx_wait(bin_)
                if with_sw:
                    scatter_wait(brn, (ph + 4) % NIDX)
                gather_start(brn, bin_)
            if with_i:
                idx_start(i + 3, (ph + 3) % NIDX)

        for j in range(NBUF):
            idx_start(j, j)
        idx_wait(0)
        gather_start(0, 0)
        step(0, 0, with_sw=False)
        step(1, 1, with_sw=False)

        def group(io, carry):
            i0 = 2 + io * NIDX
            for u in range(NIDX):
                step(i0 + u, (2 + u) % NIDX)
            return carry

        lax.fori_loop(0, (CPT - 8) // NIDX, group, None)

        step(74, 2)
        step(75, 3)
        step(76, 4)
        step(77, 5, with_i=False)
        step(78, 0, with_i=False)
        step(79, 1, with_g=False, with_i=False)
        scatter_wait(2, 5)
        scatter_wait(0, 0)
        scatter_wait(1, 1)

        plsc.subcore_barrier()
        pltpu.sync_copy(
            agg_sh.at[pl.ds(sid * ROWS_PER_TILE, ROWS_PER_TILE)],
            out_hbm.at[cid, pl.ds(sid * ROWS_PER_TILE, ROWS_PER_TILE)],
        )

        @pl.when(sid == 0)
        def _():
            pltpu.sync_copy(
                agg_sh.at[pl.ds(TAIL_BASE, TAIL)],
                out_hbm.at[cid, pl.ds(TAIL_BASE, TAIL)],
            )

    return k(support, src, dst, zeros)


def _mm_bias_relu(parts, w, b, n_out):
    def body(p_ref, w_ref, b_ref, o_ref):
        agg = p_ref[0] + p_ref[1]
        o_ref[...] = jnp.maximum(
            jnp.dot(agg, w_ref[...], preferred_element_type=jnp.float32) + b_ref[...],
            0.0,
        )

    d = parts.shape[2]
    return pl.pallas_call(
        body,
        grid=(GRID,),
        in_specs=[
            pl.BlockSpec((NC, ROW_BLK, d), lambda i: (0, i, 0)),
            pl.BlockSpec(w.shape, lambda i: (0, 0)),
            pl.BlockSpec((1, n_out), lambda i: (0, 0)),
        ],
        out_specs=pl.BlockSpec((ROW_BLK, n_out), lambda i: (i, 0)),
        out_shape=jax.ShapeDtypeStruct((N_NODES, n_out), jnp.float32),
    )(parts, w, b.reshape(1, n_out))


def _mm_bias_log_softmax(parts, w, b, n_out):
    def body(p_ref, w_ref, b_ref, o_ref):
        agg = p_ref[0] + p_ref[1]
        o = jnp.dot(agg, w_ref[...], preferred_element_type=jnp.float32) + b_ref[...]
        m = jnp.max(o, axis=1, keepdims=True)
        e = jnp.exp(o - m)
        s = jnp.sum(e, axis=1, keepdims=True)
        o_ref[...] = o - m - jnp.log(s)

    d = parts.shape[2]
    return pl.pallas_call(
        body,
        grid=(GRID,),
        in_specs=[
            pl.BlockSpec((NC, ROW_BLK, d), lambda i: (0, i, 0)),
            pl.BlockSpec(w.shape, lambda i: (0, 0)),
            pl.BlockSpec((1, n_out), lambda i: (0, 0)),
        ],
        out_specs=pl.BlockSpec((ROW_BLK, n_out), lambda i: (i, 0)),
        out_shape=jax.ShapeDtypeStruct((N_NODES, n_out), jnp.float32),
    )(parts, w, b.reshape(1, n_out))


def kernel(x, adjs, W0, b0, W1, b1):
    pad = E_PAD - N_EDGES
    src = jnp.concatenate([adjs[0].astype(jnp.int32), jnp.zeros((pad,), jnp.int32)])
    dst = jnp.concatenate(
        [adjs[1].astype(jnp.int32),
         N_NODES + (jnp.arange(pad, dtype=jnp.int32) % 16)]
    )
    nfeat = x.shape[1]
    nhid = W0.shape[1]
    ncls = W1.shape[1]
    z = jnp.zeros((ROWS_PER_TILE, nfeat), jnp.float32)

    parts1 = _seg_sum_partials(x, src, dst, z, nfeat)
    h = _mm_bias_relu(parts1, W0, b0, nhid)
    parts2 = _seg_sum_partials(h, src, dst, z, nhid)
    return _mm_bias_log_softmax(parts2, W1, b1, ncls)


# speedup vs baseline: 3.3047x; 3.2830x over previous
import functools

import jax
import jax.numpy as jnp
from jax import lax
from jax.experimental import pallas as pl
from jax.experimental.pallas import tpu as pltpu
from jax.experimental.pallas import tpu_sc as plsc

N_NODES = 10000
N_EDGES = 320000
NC = 2
NS = 16
NW = NC * NS
CHUNK = 128
N_CHUNKS = N_EDGES // CHUNK
ITERS = -(-N_CHUNKS // NW)
ROWS_PER_TILE = (N_NODES // NS) // 8 * 8
TAIL_BASE = ROWS_PER_TILE * NS
TAIL = N_NODES - TAIL_BASE
CPT = 80
E_PAD = NW * CPT * CHUNK
AGG_ROWS = N_NODES + 16
NBUF = 3
NIDX = 6

ROW_BLK = 1000
GRID = N_NODES // ROW_BLK


def _seg_sum_partials(support, src, dst, zeros, d):
    mesh = plsc.VectorSubcoreMesh(
        core_axis_name="c", subcore_axis_name="s", num_cores=NC, num_subcores=NS
    )

    @functools.partial(
        pl.kernel,
        out_type=jax.ShapeDtypeStruct((NC, N_NODES, d), jnp.float32),
        mesh=mesh,
        scratch_types=[
            [pltpu.VMEM((CHUNK,), jnp.int32) for _ in range(NIDX)],
            [pltpu.VMEM((CHUNK,), jnp.int32) for _ in range(NIDX)],
            [pltpu.VMEM((CHUNK, d), jnp.float32) for _ in range(NBUF)],
            pltpu.VMEM_SHARED((AGG_ROWS, d), jnp.float32),
            [pltpu.SemaphoreType.DMA for _ in range(NIDX)],
            [pltpu.SemaphoreType.DMA for _ in range(NBUF)],
            [pltpu.SemaphoreType.DMA for _ in range(NBUF)],
        ],
    )
    def k(support_hbm, src_hbm, dst_hbm, zeros_hbm, out_hbm,
          src_idx, dst_idx, rows, agg_sh, isem, gsem, ssem):
        cid = lax.axis_index("c")
        sid = lax.axis_index("s")
        wid = sid * NC + cid

        pltpu.sync_copy(zeros_hbm, agg_sh.at[pl.ds(sid * ROWS_PER_TILE, ROWS_PER_TILE)])

        @pl.when(sid == 0)
        def _():
            pltpu.sync_copy(
                zeros_hbm.at[pl.ds(0, AGG_ROWS - TAIL_BASE)],
                agg_sh.at[pl.ds(TAIL_BASE, AGG_ROWS - TAIL_BASE)],
            )

        plsc.subcore_barrier()

        def gather_start(i, b):
            pltpu.async_copy(support_hbm.at[src_idx.at[i]], rows[b], gsem[b])

        def gather_wait(i, b):
            pltpu.make_async_copy(support_hbm.at[src_idx.at[i]], rows[b], gsem[b]).wait()

        def scatter_start(i, b):
            pltpu.async_copy(rows[b], agg_sh.at[dst_idx.at[i]], ssem[b], add=True)

        def scatter_wait(i, b):
            pltpu.make_async_copy(rows[b], agg_sh.at[dst_idx.at[i]], ssem[b]).wait()

        def idx_start(i, bi):
            base = (wid * CPT + i) * CHUNK
            pltpu.async_copy(src_hbm.at[pl.ds(base, CHUNK)], src_idx[bi], isem[bi])
            pltpu.async_copy(dst_hbm.at[pl.ds(base, CHUNK)], dst_idx[bi], isem[bi])

        def idx_wait(bi):
            pltpu.make_async_copy(src_hbm.at[pl.ds(0, CHUNK)], src_idx[bi], isem[bi]).wait()
            pltpu.make_async_copy(dst_hbm.at[pl.ds(0, CHUNK)], dst_idx[bi], isem[bi]).wait()

        def gather_start(br, bi):
            pltpu.async_copy(support_hbm.at[src_idx[bi]], rows[br], gsem[br])

        def gather_wait(br, bi):
            pltpu.make_async_copy(support_hbm.at[src_idx[bi]], rows[br], gsem[br]).wait()

        def scatter_start(br, bi):
            pltpu.async_copy(rows[br], agg_sh.at[dst_idx[bi]], ssem[br], add=True)

        def scatter_wait(br, bi):
            pltpu.make_async_copy(rows[br], agg_sh.at[dst_idx[bi]], ssem[br]).wait()

        def step(i, ph, with_g=True, with_i=True, with_sw=True):
            br, bi = ph % NBUF, ph % NIDX
            gather_wait(br, bi)
            scatter_start(br, bi)
            if with_g:
                brn, bin_ = (ph + 1) % NBUF, (ph + 1) % NIDX
                idx_wait(bin_)
                if with_sw:
                    scatter_wait(brn, (ph + 4) % NIDX)
                gather_start(brn, bin_)
            if with_i:
                idx_start(i + 3, (ph + 3) % NIDX)

        for j in range(NBUF):
            idx_start(j, j)
        idx_wait(0)
        gather_start(0, 0)
        step(0, 0, with_sw=False)
        step(1, 1, with_sw=False)

        def group(io, carry):
            i0 = 2 + io * NIDX
            for u in range(NIDX):
                step(i0 + u, (2 + u) % NIDX)
            return carry

        lax.fori_loop(0, (CPT - 8) // NIDX, group, None)

        step(74, 2)
        step(75, 3)
        step(76, 4)
        step(77, 5, with_i=False)
        step(78, 0, with_i=False)
        step(79, 1, with_g=False, with_i=False)
        scatter_wait(2, 5)
        scatter_wait(0, 0)
        scatter_wait(1, 1)

        plsc.subcore_barrier()
        pltpu.sync_copy(
            agg_sh.at[pl.ds(sid * ROWS_PER_TILE, ROWS_PER_TILE)],
            out_hbm.at[cid, pl.ds(sid * ROWS_PER_TILE, ROWS_PER_TILE)],
        )

        @pl.when(sid == 0)
        def _():
            pltpu.sync_copy(
                agg_sh.at[pl.ds(TAIL_BASE, TAIL)],
                out_hbm.at[cid, pl.ds(TAIL_BASE, TAIL)],
            )

    return k(support, src, dst, zeros)


def _mm_bias_relu(parts, w, b, n_out):
    def body(p_ref, w_ref, b_ref, o_ref):
        agg = p_ref[0] + p_ref[1]
        o_ref[...] = jnp.maximum(
            jnp.dot(agg, w_ref[...], preferred_element_type=jnp.float32) + b_ref[...],
            0.0,
        )

    d = parts.shape[2]
    return pl.pallas_call(
        body,
        grid=(GRID,),
        in_specs=[
            pl.BlockSpec((NC, ROW_BLK, d), lambda i: (0, i, 0)),
            pl.BlockSpec(w.shape, lambda i: (0, 0)),
            pl.BlockSpec((1, n_out), lambda i: (0, 0)),
        ],
        out_specs=pl.BlockSpec((ROW_BLK, n_out), lambda i: (i, 0)),
        out_shape=jax.ShapeDtypeStruct((N_NODES, n_out), jnp.float32),
    )(parts, w, b.reshape(1, n_out))


def _mm_bias_log_softmax(parts, w, b, n_out):
    def body(p_ref, w_ref, b_ref, o_ref):
        agg = p_ref[0] + p_ref[1]
        o = jnp.dot(agg, w_ref[...], preferred_element_type=jnp.float32) + b_ref[...]
        m = jnp.max(o, axis=1, keepdims=True)
        e = jnp.exp(o - m)
        s = jnp.sum(e, axis=1, keepdims=True)
        o_ref[...] = o - m - jnp.log(s)

    d = parts.shape[2]
    return pl.pallas_call(
        body,
        grid=(GRID,),
        in_specs=[
            pl.BlockSpec((NC, ROW_BLK, d), lambda i: (0, i, 0)),
            pl.BlockSpec(w.shape, lambda i: (0, 0)),
            pl.BlockSpec((1, n_out), lambda i: (0, 0)),
        ],
        out_specs=pl.BlockSpec((ROW_BLK, n_out), lambda i: (i, 0)),
        out_shape=jax.ShapeDtypeStruct((N_NODES, n_out), jnp.float32),
    )(parts, w, b.reshape(1, n_out))


def kernel(x, adjs, W0, b0, W1, b1):
    pad = E_PAD - N_EDGES
    src = jnp.concatenate(
        [adjs[0].astype(jnp.int32),
         jnp.arange(pad, dtype=jnp.int32) * 997 % N_NODES]
    )
    dst = jnp.concatenate(
        [adjs[1].astype(jnp.int32),
         N_NODES + (jnp.arange(pad, dtype=jnp.int32) % 16)]
    )
    nfeat = x.shape[1]
    nhid = W0.shape[1]
    ncls = W1.shape[1]
    z = jnp.zeros((ROWS_PER_TILE, nfeat), jnp.float32)

    parts1 = _seg_sum_partials(x, src, dst, z, nfeat)
    h = _mm_bias_relu(parts1, W0, b0, nhid)
    parts2 = _seg_sum_partials(h, src, dst, z, nhid)
    return _mm_bias_log_softmax(parts2, W1, b1, ncls)


# speedup vs baseline: 3.5727x; 1.0811x over previous
import functools

import jax
import jax.numpy as jnp
from jax import lax
from jax.experimental import pallas as pl
from jax.experimental.pallas import tpu as pltpu
from jax.experimental.pallas import tpu_sc as plsc

N_NODES = 10000
N_EDGES = 320000
NC = 2
NS = 16
NW = NC * NS
CHUNK = 128
N_CHUNKS = N_EDGES // CHUNK
ITERS = -(-N_CHUNKS // NW)
ROWS_PER_TILE = (N_NODES // NS) // 8 * 8
TAIL_BASE = ROWS_PER_TILE * NS
TAIL = N_NODES - TAIL_BASE
CPT = 80
E_PAD = NW * CPT * CHUNK
AGG_ROWS = N_NODES + 16
NBUF = 3
NIDX = 6

ROW_BLK = 1000
GRID = N_NODES // ROW_BLK


def _seg_sum_partials(support, src, dst, zeros, d, linear_tiling=False):
    mesh = plsc.VectorSubcoreMesh(
        core_axis_name="c", subcore_axis_name="s", num_cores=NC, num_subcores=NS
    )
    params = (
        pltpu.CompilerParams(use_tc_tiling_on_sc=False) if linear_tiling else None
    )

    @functools.partial(
        pl.kernel,
        compiler_params=params,
        out_type=jax.ShapeDtypeStruct((NC, N_NODES, d), jnp.float32),
        mesh=mesh,
        scratch_types=[
            [pltpu.VMEM((CHUNK,), jnp.int32) for _ in range(NIDX)],
            [pltpu.VMEM((CHUNK,), jnp.int32) for _ in range(NIDX)],
            [pltpu.VMEM((CHUNK, d), jnp.float32) for _ in range(NBUF)],
            pltpu.VMEM_SHARED((AGG_ROWS, d), jnp.float32),
            [pltpu.SemaphoreType.DMA for _ in range(NIDX)],
            [pltpu.SemaphoreType.DMA for _ in range(NBUF)],
            [pltpu.SemaphoreType.DMA for _ in range(NBUF)],
        ],
    )
    def k(support_hbm, src_hbm, dst_hbm, zeros_hbm, out_hbm,
          src_idx, dst_idx, rows, agg_sh, isem, gsem, ssem):
        cid = lax.axis_index("c")
        sid = lax.axis_index("s")
        wid = sid * NC + cid

        pltpu.sync_copy(zeros_hbm, agg_sh.at[pl.ds(sid * ROWS_PER_TILE, ROWS_PER_TILE)])

        @pl.when(sid == 0)
        def _():
            pltpu.sync_copy(
                zeros_hbm.at[pl.ds(0, AGG_ROWS - TAIL_BASE)],
                agg_sh.at[pl.ds(TAIL_BASE, AGG_ROWS - TAIL_BASE)],
            )

        plsc.subcore_barrier()

        def gather_start(i, b):
            pltpu.async_copy(support_hbm.at[src_idx.at[i]], rows[b], gsem[b])

        def gather_wait(i, b):
            pltpu.make_async_copy(support_hbm.at[src_idx.at[i]], rows[b], gsem[b]).wait()

        def scatter_start(i, b):
            pltpu.async_copy(rows[b], agg_sh.at[dst_idx.at[i]], ssem[b], add=True)

        def scatter_wait(i, b):
            pltpu.make_async_copy(rows[b], agg_sh.at[dst_idx.at[i]], ssem[b]).wait()

        def idx_start(i, bi):
            base = (wid * CPT + i) * CHUNK
            pltpu.async_copy(src_hbm.at[pl.ds(base, CHUNK)], src_idx[bi], isem[bi])
            pltpu.async_copy(dst_hbm.at[pl.ds(base, CHUNK)], dst_idx[bi], isem[bi])

        def idx_wait(bi):
            pltpu.make_async_copy(src_hbm.at[pl.ds(0, CHUNK)], src_idx[bi], isem[bi]).wait()
            pltpu.make_async_copy(dst_hbm.at[pl.ds(0, CHUNK)], dst_idx[bi], isem[bi]).wait()

        def gather_start(br, bi):
            pltpu.async_copy(support_hbm.at[src_idx[bi]], rows[br], gsem[br])

        def gather_wait(br, bi):
            pltpu.make_async_copy(support_hbm.at[src_idx[bi]], rows[br], gsem[br]).wait()

        def scatter_start(br, bi):
            pltpu.async_copy(rows[br], agg_sh.at[dst_idx[bi]], ssem[br], add=True)

        def scatter_wait(br, bi):
            pltpu.make_async_copy(rows[br], agg_sh.at[dst_idx[bi]], ssem[br]).wait()

        def step(i, ph, with_g=True, with_i=True, with_sw=True):
            br, bi = ph % NBUF, ph % NIDX
            gather_wait(br, bi)
            scatter_start(br, bi)
            if with_g:
                brn, bin_ = (ph + 1) % NBUF, (ph + 1) % NIDX
                idx_wait(bin_)
                if with_sw:
                    scatter_wait(brn, (ph + 4) % NIDX)
                gather_start(brn, bin_)
            if with_i:
                idx_start(i + 3, (ph + 3) % NIDX)

        for j in range(NBUF):
            idx_start(j, j)
        idx_wait(0)
        gather_start(0, 0)
        step(0, 0, with_sw=False)
        step(1, 1, with_sw=False)

        def group(io, carry):
            i0 = 2 + io * NIDX
            for u in range(NIDX):
                step(i0 + u, (2 + u) % NIDX)
            return carry

        lax.fori_loop(0, (CPT - 8) // NIDX, group, None)

        step(74, 2)
        step(75, 3)
        step(76, 4)
        step(77, 5, with_i=False)
        step(78, 0, with_i=False)
        step(79, 1, with_g=False, with_i=False)
        scatter_wait(2, 5)
        scatter_wait(0, 0)
        scatter_wait(1, 1)

        plsc.subcore_barrier()
        pltpu.sync_copy(
            agg_sh.at[pl.ds(sid * ROWS_PER_TILE, ROWS_PER_TILE)],
            out_hbm.at[cid, pl.ds(sid * ROWS_PER_TILE, ROWS_PER_TILE)],
        )

        @pl.when(sid == 0)
        def _():
            pltpu.sync_copy(
                agg_sh.at[pl.ds(TAIL_BASE, TAIL)],
                out_hbm.at[cid, pl.ds(TAIL_BASE, TAIL)],
            )

    return k(support, src, dst, zeros)


def _layer1_tc(parts, w0, b0, w1, nhid, ncls):
    def body(p_ref, w0_ref, b0_ref, w1_ref, o_ref):
        agg = p_ref[0] + p_ref[1]
        h = jnp.maximum(
            jnp.dot(agg, w0_ref[...], preferred_element_type=jnp.float32) + b0_ref[...],
            0.0,
        )
        o_ref[...] = jnp.dot(h, w1_ref[...], preferred_element_type=jnp.float32)

    d = parts.shape[2]
    return pl.pallas_call(
        body,
        grid=(GRID,),
        in_specs=[
            pl.BlockSpec((NC, ROW_BLK, d), lambda i: (0, i, 0)),
            pl.BlockSpec(w0.shape, lambda i: (0, 0)),
            pl.BlockSpec((1, nhid), lambda i: (0, 0)),
            pl.BlockSpec(w1.shape, lambda i: (0, 0)),
        ],
        out_specs=pl.BlockSpec((ROW_BLK, ncls), lambda i: (i, 0)),
        out_shape=jax.ShapeDtypeStruct((N_NODES, ncls), jnp.float32),
    )(parts, w0, b0.reshape(1, nhid), w1)


def _bias_log_softmax(parts, b, n_out):
    def body(p_ref, b_ref, o_ref):
        o = p_ref[0] + p_ref[1] + b_ref[...]
        m = jnp.max(o, axis=1, keepdims=True)
        e = jnp.exp(o - m)
        s = jnp.sum(e, axis=1, keepdims=True)
        o_ref[...] = o - m - jnp.log(s)

    return pl.pallas_call(
        body,
        grid=(GRID,),
        in_specs=[
            pl.BlockSpec((NC, ROW_BLK, n_out), lambda i: (0, i, 0)),
            pl.BlockSpec((1, n_out), lambda i: (0, 0)),
        ],
        out_specs=pl.BlockSpec((ROW_BLK, n_out), lambda i: (i, 0)),
        out_shape=jax.ShapeDtypeStruct((N_NODES, n_out), jnp.float32),
    )(parts, b.reshape(1, n_out))


def kernel(x, adjs, W0, b0, W1, b1):
    pad = E_PAD - N_EDGES
    src = jnp.concatenate(
        [adjs[0].astype(jnp.int32),
         jnp.arange(pad, dtype=jnp.int32) * 997 % N_NODES]
    )
    dst = jnp.concatenate(
        [adjs[1].astype(jnp.int32),
         N_NODES + (jnp.arange(pad, dtype=jnp.int32) % 16)]
    )
    nfeat = x.shape[1]
    nhid = W0.shape[1]
    ncls = W1.shape[1]
    z128 = jnp.zeros((ROWS_PER_TILE, nfeat), jnp.float32)
    z64 = jnp.zeros((ROWS_PER_TILE, ncls), jnp.float32)

    parts1 = _seg_sum_partials(x, src, dst, z128, nfeat)
    s2 = _layer1_tc(parts1, W0, b0, W1, nhid, ncls)
    parts2 = _seg_sum_partials(s2, src, dst, z64, ncls, linear_tiling=True)
    return _bias_log_softmax(parts2, b1, ncls)


# speedup vs baseline: 3.9913x; 1.1172x over previous
import functools

import jax
import jax.numpy as jnp
from jax import lax
from jax.experimental import pallas as pl
from jax.experimental.pallas import tpu as pltpu
from jax.experimental.pallas import tpu_sc as plsc

N_NODES = 10000
N_EDGES = 320000
NC = 2
NS = 16
NW = NC * NS
CHUNK = 128
N_CHUNKS = N_EDGES // CHUNK
ITERS = -(-N_CHUNKS // NW)
ROWS_PER_TILE = (N_NODES // NS) // 8 * 8
TAIL_BASE = ROWS_PER_TILE * NS
TAIL = N_NODES - TAIL_BASE
CPT = 80
E_PAD = NW * CPT * CHUNK
AGG_ROWS = N_NODES + 16
NBUF = 3
NIDX = 6

ROW_BLK = 1000
GRID = N_NODES // ROW_BLK


def _seg_sum_partials(support, src, dst, zeros, d, linear_tiling=False):
    mesh = plsc.VectorSubcoreMesh(
        core_axis_name="c", subcore_axis_name="s", num_cores=NC, num_subcores=NS
    )
    params = (
        pltpu.CompilerParams(use_tc_tiling_on_sc=False) if linear_tiling else None
    )

    @functools.partial(
        pl.kernel,
        compiler_params=params,
        out_type=jax.ShapeDtypeStruct((NC, N_NODES, d), jnp.float32),
        mesh=mesh,
        scratch_types=[
            [pltpu.VMEM((CHUNK,), jnp.int32) for _ in range(NIDX)],
            [pltpu.VMEM((CHUNK,), jnp.int32) for _ in range(NIDX)],
            [pltpu.VMEM((CHUNK, d), jnp.float32) for _ in range(NBUF)],
            pltpu.VMEM_SHARED((AGG_ROWS, d), jnp.float32),
            [pltpu.SemaphoreType.DMA for _ in range(NIDX)],
            [pltpu.SemaphoreType.DMA for _ in range(NBUF)],
            [pltpu.SemaphoreType.DMA for _ in range(NBUF)],
        ],
    )
    def k(support_hbm, src_hbm, dst_hbm, zeros_hbm, out_hbm,
          src_idx, dst_idx, rows, agg_sh, isem, gsem, ssem):
        cid = lax.axis_index("c")
        sid = lax.axis_index("s")
        wid = sid * NC + cid

        pltpu.sync_copy(zeros_hbm, agg_sh.at[pl.ds(sid * ROWS_PER_TILE, ROWS_PER_TILE)])

        @pl.when(sid == 0)
        def _():
            pltpu.sync_copy(
                zeros_hbm.at[pl.ds(0, AGG_ROWS - TAIL_BASE)],
                agg_sh.at[pl.ds(TAIL_BASE, AGG_ROWS - TAIL_BASE)],
            )

        plsc.subcore_barrier()

        def gather_start(i, b):
            pltpu.async_copy(support_hbm.at[src_idx.at[i]], rows[b], gsem[b])

        def gather_wait(i, b):
            pltpu.make_async_copy(support_hbm.at[src_idx.at[i]], rows[b], gsem[b]).wait()

        def scatter_start(i, b):
            pltpu.async_copy(rows[b], agg_sh.at[dst_idx.at[i]], ssem[b], add=True)

        def scatter_wait(i, b):
            pltpu.make_async_copy(rows[b], agg_sh.at[dst_idx.at[i]], ssem[b]).wait()

        def idx_start(i, bi):
            base = (wid * CPT + i) * CHUNK
            pltpu.async_copy(src_hbm.at[pl.ds(base, CHUNK)], src_idx[bi], isem[bi])
            pltpu.async_copy(dst_hbm.at[pl.ds(base, CHUNK)], dst_idx[bi], isem[bi])

        def idx_wait(bi):
            pltpu.make_async_copy(src_hbm.at[pl.ds(0, CHUNK)], src_idx[bi], isem[bi]).wait()
            pltpu.make_async_copy(dst_hbm.at[pl.ds(0, CHUNK)], dst_idx[bi], isem[bi]).wait()

        def gather_start(br, bi):
            pltpu.async_copy(support_hbm.at[src_idx[bi]], rows[br], gsem[br])

        def gather_wait(br, bi):
            pltpu.make_async_copy(support_hbm.at[src_idx[bi]], rows[br], gsem[br]).wait()

        def scatter_start(br, bi):
            pltpu.async_copy(rows[br], agg_sh.at[dst_idx[bi]], ssem[br], add=True)

        def scatter_wait(br, bi):
            pltpu.make_async_copy(rows[br], agg_sh.at[dst_idx[bi]], ssem[br]).wait()

        def step(i, ph, with_g=True, with_i=True, with_sw=True):
            br, bi = ph % NBUF, ph % NIDX
            gather_wait(br, bi)
            scatter_start(br, bi)
            if with_g:
                brn, bin_ = (ph + 1) % NBUF, (ph + 1) % NIDX
                idx_wait(bin_)
                if with_sw:
                    scatter_wait(brn, (ph + 4) % NIDX)
                gather_start(brn, bin_)
            if with_i:
                idx_start(i + 3, (ph + 3) % NIDX)

        for j in range(NBUF):
            idx_start(j, j)
        idx_wait(0)
        gather_start(0, 0)
        step(0, 0, with_sw=False)
        step(1, 1, with_sw=False)

        def group(io, carry):
            i0 = 2 + io * NIDX
            for u in range(NIDX):
                step(i0 + u, (2 + u) % NIDX)
            return carry

        lax.fori_loop(0, (CPT - 8) // NIDX, group, None)

        step(74, 2)
        step(75, 3)
        step(76, 4)
        step(77, 5, with_i=False)
        step(78, 0, with_i=False)
        step(79, 1, with_g=False, with_i=False)
        scatter_wait(2, 5)
        scatter_wait(0, 0)
        scatter_wait(1, 1)

        plsc.subcore_barrier()
        pltpu.sync_copy(
            agg_sh.at[pl.ds(sid * ROWS_PER_TILE, ROWS_PER_TILE)],
            out_hbm.at[cid, pl.ds(sid * ROWS_PER_TILE, ROWS_PER_TILE)],
        )

        @pl.when(sid == 0)
        def _():
            pltpu.sync_copy(
                agg_sh.at[pl.ds(TAIL_BASE, TAIL)],
                out_hbm.at[cid, pl.ds(TAIL_BASE, TAIL)],
            )

    return k(support, src, dst, zeros)


GSZ = 8
NGRP = CPT // GSZ


def _seg_sum_partials_grouped(support, src2d, dst2d, zeros, d):
    mesh = plsc.VectorSubcoreMesh(
        core_axis_name="c", subcore_axis_name="s", num_cores=NC, num_subcores=NS
    )

    @functools.partial(
        pl.kernel,
        compiler_params=pltpu.CompilerParams(use_tc_tiling_on_sc=False),
        out_type=jax.ShapeDtypeStruct((NC, N_NODES, d), jnp.float32),
        mesh=mesh,
        scratch_types=[
            [pltpu.VMEM((GSZ, CHUNK), jnp.int32) for _ in range(2)],
            [pltpu.VMEM((GSZ, CHUNK), jnp.int32) for _ in range(2)],
            [pltpu.VMEM((CHUNK, d), jnp.float32) for _ in range(GSZ)],
            pltpu.VMEM_SHARED((AGG_ROWS, d), jnp.float32),
            [pltpu.SemaphoreType.DMA for _ in range(2)],
            [pltpu.SemaphoreType.DMA for _ in range(GSZ)],
            [pltpu.SemaphoreType.DMA for _ in range(GSZ)],
        ],
    )
    def k(support_hbm, src_hbm, dst_hbm, zeros_hbm, out_hbm,
          src_idx, dst_idx, rows, agg_sh, isem, gsem, ssem):
        cid = lax.axis_index("c")
        sid = lax.axis_index("s")
        wid = sid * NC + cid

        pltpu.sync_copy(zeros_hbm, agg_sh.at[pl.ds(sid * ROWS_PER_TILE, ROWS_PER_TILE)])

        @pl.when(sid == 0)
        def _():
            pltpu.sync_copy(
                zeros_hbm.at[pl.ds(0, AGG_ROWS - TAIL_BASE)],
                agg_sh.at[pl.ds(TAIL_BASE, AGG_ROWS - TAIL_BASE)],
            )

        plsc.subcore_barrier()

        def idx_start(g, pg):
            row0 = wid * CPT + g * GSZ
            pltpu.async_copy(src_hbm.at[pl.ds(row0, GSZ)], src_idx[pg], isem[pg])
            pltpu.async_copy(dst_hbm.at[pl.ds(row0, GSZ)], dst_idx[pg], isem[pg])

        def idx_wait(pg):
            pltpu.make_async_copy(src_hbm.at[pl.ds(0, GSZ)], src_idx[pg], isem[pg]).wait()
            pltpu.make_async_copy(dst_hbm.at[pl.ds(0, GSZ)], dst_idx[pg], isem[pg]).wait()

        def group(g, pg, first=False, last=False):
            idx_wait(pg)
            for u in range(GSZ):
                if not first:
                    pltpu.make_async_copy(
                        rows[u], agg_sh.at[dst_idx[1 - pg].at[u]], ssem[u]
                    ).wait()
                pltpu.async_copy(support_hbm.at[src_idx[pg].at[u]], rows[u], gsem[u])
            if not last:
                idx_start(g + 1, 1 - pg)
            for u in range(GSZ):
                pltpu.make_async_copy(
                    support_hbm.at[src_idx[pg].at[u]], rows[u], gsem[u]
                ).wait()
                pltpu.async_copy(
                    rows[u], agg_sh.at[dst_idx[pg].at[u]], ssem[u], add=True
                )

        idx_start(0, 0)
        group(0, 0, first=True)

        def two_groups(jo, carry):
            g = 1 + 2 * jo
            group(g, 1)
            group(g + 1, 0)
            return carry

        lax.fori_loop(0, (NGRP - 2) // 2, two_groups, None)

        group(NGRP - 1, (NGRP - 1) % 2, last=True)
        for u in range(GSZ):
            pltpu.make_async_copy(
                rows[u], agg_sh.at[dst_idx[(NGRP - 1) % 2].at[u]], ssem[u]
            ).wait()

        plsc.subcore_barrier()
        pltpu.sync_copy(
            agg_sh.at[pl.ds(sid * ROWS_PER_TILE, ROWS_PER_TILE)],
            out_hbm.at[cid, pl.ds(sid * ROWS_PER_TILE, ROWS_PER_TILE)],
        )

        @pl.when(sid == 0)
        def _():
            pltpu.sync_copy(
                agg_sh.at[pl.ds(TAIL_BASE, TAIL)],
                out_hbm.at[cid, pl.ds(TAIL_BASE, TAIL)],
            )

    return k(support, src2d, dst2d, zeros)


def _layer1_tc(parts, w0, b0, w1, nhid, ncls):
    def body(p_ref, w0_ref, b0_ref, w1_ref, o_ref):
        agg = p_ref[0] + p_ref[1]
        h = jnp.maximum(
            jnp.dot(agg, w0_ref[...], preferred_element_type=jnp.float32) + b0_ref[...],
            0.0,
        )
        o_ref[...] = jnp.dot(h, w1_ref[...], preferred_element_type=jnp.float32)

    d = parts.shape[2]
    return pl.pallas_call(
        body,
        grid=(GRID,),
        in_specs=[
            pl.BlockSpec((NC, ROW_BLK, d), lambda i: (0, i, 0)),
            pl.BlockSpec(w0.shape, lambda i: (0, 0)),
            pl.BlockSpec((1, nhid), lambda i: (0, 0)),
            pl.BlockSpec(w1.shape, lambda i: (0, 0)),
        ],
        out_specs=pl.BlockSpec((ROW_BLK, ncls), lambda i: (i, 0)),
        out_shape=jax.ShapeDtypeStruct((N_NODES, ncls), jnp.float32),
    )(parts, w0, b0.reshape(1, nhid), w1)


def _bias_log_softmax(parts, b, n_out):
    def body(p_ref, b_ref, o_ref):
        o = p_ref[0] + p_ref[1] + b_ref[...]
        m = jnp.max(o, axis=1, keepdims=True)
        e = jnp.exp(o - m)
        s = jnp.sum(e, axis=1, keepdims=True)
        o_ref[...] = o - m - jnp.log(s)

    return pl.pallas_call(
        body,
        grid=(GRID,),
        in_specs=[
            pl.BlockSpec((NC, ROW_BLK, n_out), lambda i: (0, i, 0)),
            pl.BlockSpec((1, n_out), lambda i: (0, 0)),
        ],
        out_specs=pl.BlockSpec((ROW_BLK, n_out), lambda i: (i, 0)),
        out_shape=jax.ShapeDtypeStruct((N_NODES, n_out), jnp.float32),
    )(parts, b.reshape(1, n_out))


def kernel(x, adjs, W0, b0, W1, b1):
    pad = E_PAD - N_EDGES
    src = jnp.concatenate(
        [adjs[0].astype(jnp.int32),
         jnp.arange(pad, dtype=jnp.int32) * 997 % N_NODES]
    )
    dst = jnp.concatenate(
        [adjs[1].astype(jnp.int32),
         N_NODES + (jnp.arange(pad, dtype=jnp.int32) % 16)]
    )
    nfeat = x.shape[1]
    nhid = W0.shape[1]
    ncls = W1.shape[1]
    z128 = jnp.zeros((ROWS_PER_TILE, nfeat), jnp.float32)
    z64 = jnp.zeros((ROWS_PER_TILE, ncls), jnp.float32)

    src2d = src.reshape(NW * CPT, CHUNK)
    dst2d = dst.reshape(NW * CPT, CHUNK)

    parts1 = _seg_sum_partials(x, src, dst, z128, nfeat)
    s2 = _layer1_tc(parts1, W0, b0, W1, nhid, ncls)
    parts2 = _seg_sum_partials_grouped(s2, src2d, dst2d, z64, ncls)
    return _bias_log_softmax(parts2, b1, ncls)


# speedup vs baseline: 4.1481x; 1.0393x over previous
import functools

import jax
import jax.numpy as jnp
from jax import lax
from jax.experimental import pallas as pl
from jax.experimental.pallas import tpu as pltpu
from jax.experimental.pallas import tpu_sc as plsc

N_NODES = 10000
N_EDGES = 320000
NC = 2
NS = 16
NW = NC * NS
CHUNK = 128
N_CHUNKS = N_EDGES // CHUNK
ITERS = -(-N_CHUNKS // NW)
ROWS_PER_TILE = (N_NODES // NS) // 8 * 8
TAIL_BASE = ROWS_PER_TILE * NS
TAIL = N_NODES - TAIL_BASE
CPT = 80
E_PAD = NW * CPT * CHUNK
CPT1 = 81
E_PAD1 = NW * CPT1 * CHUNK
AGG_ROWS = N_NODES + 16
NBUF = 3
NIDX = 6

ROW_BLK = 1000
GRID = N_NODES // ROW_BLK


def _seg_sum_partials(support, src, dst, zeros, d, linear_tiling=False):
    mesh = plsc.VectorSubcoreMesh(
        core_axis_name="c", subcore_axis_name="s", num_cores=NC, num_subcores=NS
    )
    params = (
        pltpu.CompilerParams(use_tc_tiling_on_sc=False) if linear_tiling else None
    )

    @functools.partial(
        pl.kernel,
        compiler_params=params,
        out_type=jax.ShapeDtypeStruct((NC, N_NODES, d), jnp.float32),
        mesh=mesh,
        scratch_types=[
            [pltpu.VMEM((CHUNK,), jnp.int32) for _ in range(NIDX)],
            [pltpu.VMEM((CHUNK,), jnp.int32) for _ in range(NIDX)],
            [pltpu.VMEM((CHUNK, d), jnp.float32) for _ in range(NBUF)],
            pltpu.VMEM_SHARED((AGG_ROWS, d), jnp.float32),
            [pltpu.SemaphoreType.DMA for _ in range(NIDX)],
            [pltpu.SemaphoreType.DMA for _ in range(NBUF)],
            [pltpu.SemaphoreType.DMA for _ in range(NBUF)],
        ],
    )
    def k(support_hbm, src_hbm, dst_hbm, zeros_hbm, out_hbm,
          src_idx, dst_idx, rows, agg_sh, isem, gsem, ssem):
        cid = lax.axis_index("c")
        sid = lax.axis_index("s")
        wid = sid * NC + cid

        pltpu.sync_copy(zeros_hbm, agg_sh.at[pl.ds(sid * ROWS_PER_TILE, ROWS_PER_TILE)])

        @pl.when(sid == 0)
        def _():
            pltpu.sync_copy(
                zeros_hbm.at[pl.ds(0, AGG_ROWS - TAIL_BASE)],
                agg_sh.at[pl.ds(TAIL_BASE, AGG_ROWS - TAIL_BASE)],
            )

        plsc.subcore_barrier()

        def gather_start(i, b):
            pltpu.async_copy(support_hbm.at[src_idx.at[i]], rows[b], gsem[b])

        def gather_wait(i, b):
            pltpu.make_async_copy(support_hbm.at[src_idx.at[i]], rows[b], gsem[b]).wait()

        def scatter_start(i, b):
            pltpu.async_copy(rows[b], agg_sh.at[dst_idx.at[i]], ssem[b], add=True)

        def scatter_wait(i, b):
            pltpu.make_async_copy(rows[b], agg_sh.at[dst_idx.at[i]], ssem[b]).wait()

        def idx_start(i, bi):
            base = (wid * CPT + i) * CHUNK
            pltpu.async_copy(src_hbm.at[pl.ds(base, CHUNK)], src_idx[bi], isem[bi])
            pltpu.async_copy(dst_hbm.at[pl.ds(base, CHUNK)], dst_idx[bi], isem[bi])

        def idx_wait(bi):
            pltpu.make_async_copy(src_hbm.at[pl.ds(0, CHUNK)], src_idx[bi], isem[bi]).wait()
            pltpu.make_async_copy(dst_hbm.at[pl.ds(0, CHUNK)], dst_idx[bi], isem[bi]).wait()

        def gather_start(br, bi):
            pltpu.async_copy(support_hbm.at[src_idx[bi]], rows[br], gsem[br])

        def gather_wait(br, bi):
            pltpu.make_async_copy(support_hbm.at[src_idx[bi]], rows[br], gsem[br]).wait()

        def scatter_start(br, bi):
            pltpu.async_copy(rows[br], agg_sh.at[dst_idx[bi]], ssem[br], add=True)

        def scatter_wait(br, bi):
            pltpu.make_async_copy(rows[br], agg_sh.at[dst_idx[bi]], ssem[br]).wait()

        def step(i, ph, with_g=True, with_i=True, with_sw=True):
            br, bi = ph % NBUF, ph % NIDX
            gather_wait(br, bi)
            scatter_start(br, bi)
            if with_g:
                brn, bin_ = (ph + 1) % NBUF, (ph + 1) % NIDX
                idx_wait(bin_)
                if with_sw:
                    scatter_wait(brn, (ph + 4) % NIDX)
                gather_start(brn, bin_)
            if with_i:
                idx_start(i + 3, (ph + 3) % NIDX)

        for j in range(NBUF):
            idx_start(j, j)
        idx_wait(0)
        gather_start(0, 0)
        step(0, 0, with_sw=False)
        step(1, 1, with_sw=False)

        def group(io, carry):
            i0 = 2 + io * NIDX
            for u in range(NIDX):
                step(i0 + u, (2 + u) % NIDX)
            return carry

        lax.fori_loop(0, (CPT - 8) // NIDX, group, None)

        step(74, 2)
        step(75, 3)
        step(76, 4)
        step(77, 5, with_i=False)
        step(78, 0, with_i=False)
        step(79, 1, with_g=False, with_i=False)
        scatter_wait(2, 5)
        scatter_wait(0, 0)
        scatter_wait(1, 1)

        plsc.subcore_barrier()
        pltpu.sync_copy(
            agg_sh.at[pl.ds(sid * ROWS_PER_TILE, ROWS_PER_TILE)],
            out_hbm.at[cid, pl.ds(sid * ROWS_PER_TILE, ROWS_PER_TILE)],
        )

        @pl.when(sid == 0)
        def _():
            pltpu.sync_copy(
                agg_sh.at[pl.ds(TAIL_BASE, TAIL)],
                out_hbm.at[cid, pl.ds(TAIL_BASE, TAIL)],
            )

    return k(support, src, dst, zeros)


def _seg_sum_partials_grouped(support, src2d, dst2d, zeros, d, gsz, cpt):
    ngrp = cpt // gsz
    mesh = plsc.VectorSubcoreMesh(
        core_axis_name="c", subcore_axis_name="s", num_cores=NC, num_subcores=NS
    )

    @functools.partial(
        pl.kernel,
        compiler_params=pltpu.CompilerParams(use_tc_tiling_on_sc=False),
        out_type=jax.ShapeDtypeStruct((NC, N_NODES, d), jnp.float32),
        mesh=mesh,
        scratch_types=[
            [pltpu.VMEM((gsz, CHUNK), jnp.int32) for _ in range(2)],
            [pltpu.VMEM((gsz, CHUNK), jnp.int32) for _ in range(2)],
            [pltpu.VMEM((CHUNK, d), jnp.float32) for _ in range(gsz)],
            pltpu.VMEM_SHARED((AGG_ROWS, d), jnp.float32),
            [pltpu.SemaphoreType.DMA for _ in range(2)],
            [pltpu.SemaphoreType.DMA for _ in range(gsz)],
            [pltpu.SemaphoreType.DMA for _ in range(gsz)],
        ],
    )
    def k(support_hbm, src_hbm, dst_hbm, zeros_hbm, out_hbm,
          src_idx, dst_idx, rows, agg_sh, isem, gsem, ssem):
        cid = lax.axis_index("c")
        sid = lax.axis_index("s")
        wid = sid * NC + cid

        pltpu.sync_copy(zeros_hbm, agg_sh.at[pl.ds(sid * ROWS_PER_TILE, ROWS_PER_TILE)])

        @pl.when(sid == 0)
        def _():
            pltpu.sync_copy(
                zeros_hbm.at[pl.ds(0, AGG_ROWS - TAIL_BASE)],
                agg_sh.at[pl.ds(TAIL_BASE, AGG_ROWS - TAIL_BASE)],
            )

        plsc.subcore_barrier()

        def idx_start(g, pg):
            row0 = wid * cpt + g * gsz
            pltpu.async_copy(src_hbm.at[pl.ds(row0, gsz)], src_idx[pg], isem[pg])
            pltpu.async_copy(dst_hbm.at[pl.ds(row0, gsz)], dst_idx[pg], isem[pg])

        def idx_wait(pg):
            pltpu.make_async_copy(src_hbm.at[pl.ds(0, gsz)], src_idx[pg], isem[pg]).wait()
            pltpu.make_async_copy(dst_hbm.at[pl.ds(0, gsz)], dst_idx[pg], isem[pg]).wait()

        def group(g, pg, first=False):
            idx_wait(pg)
            for u in range(gsz):
                if not first:
                    pltpu.make_async_copy(
                        rows[u], agg_sh.at[dst_idx[1 - pg].at[u]], ssem[u]
                    ).wait()
                pltpu.async_copy(support_hbm.at[src_idx[pg].at[u]], rows[u], gsem[u])
            if isinstance(g, int):
                if g + 1 < ngrp:
                    idx_start(g + 1, 1 - pg)
            else:
                @pl.when(g + 1 < ngrp)
                def _():
                    idx_start(g + 1, 1 - pg)
            for u in range(gsz):
                pltpu.make_async_copy(
                    support_hbm.at[src_idx[pg].at[u]], rows[u], gsem[u]
                ).wait()
                pltpu.async_copy(
                    rows[u], agg_sh.at[dst_idx[pg].at[u]], ssem[u], add=True
                )

        idx_start(0, 0)
        group(0, 0, first=True)

        def two_groups(jo, carry):
            g = 1 + 2 * jo
            group(g, 1)
            group(g + 1, 0)
            return carry

        lax.fori_loop(0, (ngrp - 1) // 2, two_groups, None)

        if ngrp % 2 == 0:
            group(ngrp - 1, (ngrp - 1) % 2)
        for u in range(gsz):
            pltpu.make_async_copy(
                rows[u], agg_sh.at[dst_idx[(ngrp - 1) % 2].at[u]], ssem[u]
            ).wait()

        plsc.subcore_barrier()
        pltpu.sync_copy(
            agg_sh.at[pl.ds(sid * ROWS_PER_TILE, ROWS_PER_TILE)],
            out_hbm.at[cid, pl.ds(sid * ROWS_PER_TILE, ROWS_PER_TILE)],
        )

        @pl.when(sid == 0)
        def _():
            pltpu.sync_copy(
                agg_sh.at[pl.ds(TAIL_BASE, TAIL)],
                out_hbm.at[cid, pl.ds(TAIL_BASE, TAIL)],
            )

    return k(support, src2d, dst2d, zeros)


def _layer1_tc(parts, w0, b0, w1, nhid, ncls):
    def body(p_ref, w0_ref, b0_ref, w1_ref, o_ref):
        agg = p_ref[0] + p_ref[1]
        h = jnp.maximum(
            jnp.dot(agg, w0_ref[...], preferred_element_type=jnp.float32) + b0_ref[...],
            0.0,
        )
        o_ref[...] = jnp.dot(h, w1_ref[...], preferred_element_type=jnp.float32)

    d = parts.shape[2]
    return pl.pallas_call(
        body,
        grid=(GRID,),
        in_specs=[
            pl.BlockSpec((NC, ROW_BLK, d), lambda i: (0, i, 0)),
            pl.BlockSpec(w0.shape, lambda i: (0, 0)),
            pl.BlockSpec((1, nhid), lambda i: (0, 0)),
            pl.BlockSpec(w1.shape, lambda i: (0, 0)),
        ],
        out_specs=pl.BlockSpec((ROW_BLK, ncls), lambda i: (i, 0)),
        out_shape=jax.ShapeDtypeStruct((N_NODES, ncls), jnp.float32),
    )(parts, w0, b0.reshape(1, nhid), w1)


def _bias_log_softmax(parts, b, n_out):
    def body(p_ref, b_ref, o_ref):
        o = p_ref[0] + p_ref[1] + b_ref[...]
        m = jnp.max(o, axis=1, keepdims=True)
        e = jnp.exp(o - m)
        s = jnp.sum(e, axis=1, keepdims=True)
        o_ref[...] = o - m - jnp.log(s)

    return pl.pallas_call(
        body,
        grid=(GRID,),
        in_specs=[
            pl.BlockSpec((NC, ROW_BLK, n_out), lambda i: (0, i, 0)),
            pl.BlockSpec((1, n_out), lambda i: (0, 0)),
        ],
        out_specs=pl.BlockSpec((ROW_BLK, n_out), lambda i: (i, 0)),
        out_shape=jax.ShapeDtypeStruct((N_NODES, n_out), jnp.float32),
    )(parts, b.reshape(1, n_out))


def kernel(x, adjs, W0, b0, W1, b1):
    src0 = adjs[0].astype(jnp.int32)
    dst0 = adjs[1].astype(jnp.int32)

    def pad_edges(n_total):
        pad = n_total - N_EDGES
        s = jnp.concatenate([src0, jnp.arange(pad, dtype=jnp.int32) * 997 % N_NODES])
        t = jnp.concatenate([dst0, N_NODES + (jnp.arange(pad, dtype=jnp.int32) % 16)])
        return s.reshape(-1, CHUNK), t.reshape(-1, CHUNK)

    nfeat = x.shape[1]
    nhid = W0.shape[1]
    ncls = W1.shape[1]
    z128 = jnp.zeros((ROWS_PER_TILE, nfeat), jnp.float32)
    z64 = jnp.zeros((ROWS_PER_TILE, ncls), jnp.float32)
    src1, dst1 = pad_edges(E_PAD1)
    src2, dst2 = pad_edges(E_PAD)

    parts1 = _seg_sum_partials_grouped(x, src1, dst1, z128, nfeat, 3, CPT1)
    s2 = _layer1_tc(parts1, W0, b0, W1, nhid, ncls)
    parts2 = _seg_sum_partials_grouped(s2, src2, dst2, z64, ncls, 8, CPT)
    return _bias_log_softmax(parts2, b1, ncls)


# speedup vs baseline: 4.2058x; 1.0139x over previous
import functools

import jax
import jax.numpy as jnp
from jax import lax
from jax.experimental import pallas as pl
from jax.experimental.pallas import tpu as pltpu
from jax.experimental.pallas import tpu_sc as plsc

N_NODES = 10000
N_EDGES = 320000
NC = 2
NS = 16
NW = NC * NS
CHUNK = 128
N_CHUNKS = N_EDGES // CHUNK
ITERS = -(-N_CHUNKS // NW)
ROWS_PER_TILE = (N_NODES // NS) // 8 * 8
TAIL_BASE = ROWS_PER_TILE * NS
TAIL = N_NODES - TAIL_BASE
CPT = 80
E_PAD = NW * CPT * CHUNK
CPT1 = 81
E_PAD1 = NW * CPT1 * CHUNK
AGG_ROWS = N_NODES + 16
NBUF = 3
NIDX = 6

ROW_BLK = 1000
GRID = N_NODES // ROW_BLK


def _seg_sum_partials(support, src, dst, zeros, d, linear_tiling=False):
    mesh = plsc.VectorSubcoreMesh(
        core_axis_name="c", subcore_axis_name="s", num_cores=NC, num_subcores=NS
    )
    params = (
        pltpu.CompilerParams(use_tc_tiling_on_sc=False) if linear_tiling else None
    )

    @functools.partial(
        pl.kernel,
        compiler_params=params,
        out_type=jax.ShapeDtypeStruct((NC, N_NODES, d), jnp.float32),
        mesh=mesh,
        scratch_types=[
            [pltpu.VMEM((CHUNK,), jnp.int32) for _ in range(NIDX)],
            [pltpu.VMEM((CHUNK,), jnp.int32) for _ in range(NIDX)],
            [pltpu.VMEM((CHUNK, d), jnp.float32) for _ in range(NBUF)],
            pltpu.VMEM_SHARED((AGG_ROWS, d), jnp.float32),
            [pltpu.SemaphoreType.DMA for _ in range(NIDX)],
            [pltpu.SemaphoreType.DMA for _ in range(NBUF)],
            [pltpu.SemaphoreType.DMA for _ in range(NBUF)],
        ],
    )
    def k(support_hbm, src_hbm, dst_hbm, zeros_hbm, out_hbm,
          src_idx, dst_idx, rows, agg_sh, isem, gsem, ssem):
        cid = lax.axis_index("c")
        sid = lax.axis_index("s")
        wid = sid * NC + cid

        pltpu.sync_copy(zeros_hbm, agg_sh.at[pl.ds(sid * ROWS_PER_TILE, ROWS_PER_TILE)])

        @pl.when(sid == 0)
        def _():
            pltpu.sync_copy(
                zeros_hbm.at[pl.ds(0, AGG_ROWS - TAIL_BASE)],
                agg_sh.at[pl.ds(TAIL_BASE, AGG_ROWS - TAIL_BASE)],
            )

        plsc.subcore_barrier()

        def gather_start(i, b):
            pltpu.async_copy(support_hbm.at[src_idx.at[i]], rows[b], gsem[b])

        def gather_wait(i, b):
            pltpu.make_async_copy(support_hbm.at[src_idx.at[i]], rows[b], gsem[b]).wait()

        def scatter_start(i, b):
            pltpu.async_copy(rows[b], agg_sh.at[dst_idx.at[i]], ssem[b], add=True)

        def scatter_wait(i, b):
            pltpu.make_async_copy(rows[b], agg_sh.at[dst_idx.at[i]], ssem[b]).wait()

        def idx_start(i, bi):
            base = (wid * CPT + i) * CHUNK
            pltpu.async_copy(src_hbm.at[pl.ds(base, CHUNK)], src_idx[bi], isem[bi])
            pltpu.async_copy(dst_hbm.at[pl.ds(base, CHUNK)], dst_idx[bi], isem[bi])

        def idx_wait(bi):
            pltpu.make_async_copy(src_hbm.at[pl.ds(0, CHUNK)], src_idx[bi], isem[bi]).wait()
            pltpu.make_async_copy(dst_hbm.at[pl.ds(0, CHUNK)], dst_idx[bi], isem[bi]).wait()

        def gather_start(br, bi):
            pltpu.async_copy(support_hbm.at[src_idx[bi]], rows[br], gsem[br])

        def gather_wait(br, bi):
            pltpu.make_async_copy(support_hbm.at[src_idx[bi]], rows[br], gsem[br]).wait()

        def scatter_start(br, bi):
            pltpu.async_copy(rows[br], agg_sh.at[dst_idx[bi]], ssem[br], add=True)

        def scatter_wait(br, bi):
            pltpu.make_async_copy(rows[br], agg_sh.at[dst_idx[bi]], ssem[br]).wait()

        def step(i, ph, with_g=True, with_i=True, with_sw=True):
            br, bi = ph % NBUF, ph % NIDX
            gather_wait(br, bi)
            scatter_start(br, bi)
            if with_g:
                brn, bin_ = (ph + 1) % NBUF, (ph + 1) % NIDX
                idx_wait(bin_)
                if with_sw:
                    scatter_wait(brn, (ph + 4) % NIDX)
                gather_start(brn, bin_)
            if with_i:
                idx_start(i + 3, (ph + 3) % NIDX)

        for j in range(NBUF):
            idx_start(j, j)
        idx_wait(0)
        gather_start(0, 0)
        step(0, 0, with_sw=False)
        step(1, 1, with_sw=False)

        def group(io, carry):
            i0 = 2 + io * NIDX
            for u in range(NIDX):
                step(i0 + u, (2 + u) % NIDX)
            return carry

        lax.fori_loop(0, (CPT - 8) // NIDX, group, None)

        step(74, 2)
        step(75, 3)
        step(76, 4)
        step(77, 5, with_i=False)
        step(78, 0, with_i=False)
        step(79, 1, with_g=False, with_i=False)
        scatter_wait(2, 5)
        scatter_wait(0, 0)
        scatter_wait(1, 1)

        plsc.subcore_barrier()
        pltpu.sync_copy(
            agg_sh.at[pl.ds(sid * ROWS_PER_TILE, ROWS_PER_TILE)],
            out_hbm.at[cid, pl.ds(sid * ROWS_PER_TILE, ROWS_PER_TILE)],
        )

        @pl.when(sid == 0)
        def _():
            pltpu.sync_copy(
                agg_sh.at[pl.ds(TAIL_BASE, TAIL)],
                out_hbm.at[cid, pl.ds(TAIL_BASE, TAIL)],
            )

    return k(support, src, dst, zeros)


def _seg_sum_partials_grouped(support, src2d, dst2d, zeros, d, gsz, cpt):
    ngrp = cpt // gsz
    mesh = plsc.VectorSubcoreMesh(
        core_axis_name="c", subcore_axis_name="s", num_cores=NC, num_subcores=NS
    )

    @functools.partial(
        pl.kernel,
        compiler_params=pltpu.CompilerParams(use_tc_tiling_on_sc=False),
        out_type=jax.ShapeDtypeStruct((NC, N_NODES, d), jnp.float32),
        mesh=mesh,
        scratch_types=[
            [pltpu.VMEM((gsz, CHUNK), jnp.int32) for _ in range(2)],
            [pltpu.VMEM((gsz, CHUNK), jnp.int32) for _ in range(2)],
            [pltpu.VMEM((CHUNK, d), jnp.float32) for _ in range(gsz)],
            pltpu.VMEM_SHARED((AGG_ROWS, d), jnp.float32),
            [pltpu.SemaphoreType.DMA for _ in range(2)],
            [pltpu.SemaphoreType.DMA for _ in range(gsz)],
            [pltpu.SemaphoreType.DMA for _ in range(gsz)],
        ],
    )
    def k(support_hbm, src_hbm, dst_hbm, zeros_hbm, out_hbm,
          src_idx, dst_idx, rows, agg_sh, isem, gsem, ssem):
        cid = lax.axis_index("c")
        sid = lax.axis_index("s")
        wid = sid * NC + cid

        def idx_start(g, pg):
            row0 = wid * cpt + g * gsz
            pltpu.async_copy(src_hbm.at[pl.ds(row0, gsz)], src_idx[pg], isem[pg])
            pltpu.async_copy(dst_hbm.at[pl.ds(row0, gsz)], dst_idx[pg], isem[pg])

        def idx_wait(pg):
            pltpu.make_async_copy(src_hbm.at[pl.ds(0, gsz)], src_idx[pg], isem[pg]).wait()
            pltpu.make_async_copy(dst_hbm.at[pl.ds(0, gsz)], dst_idx[pg], isem[pg]).wait()

        def group(g, pg, first=False):
            idx_wait(pg)
            for u in range(gsz):
                if not first:
                    pltpu.make_async_copy(
                        rows[u], agg_sh.at[dst_idx[1 - pg].at[u]], ssem[u]
                    ).wait()
                pltpu.async_copy(support_hbm.at[src_idx[pg].at[u]], rows[u], gsem[u])
            if isinstance(g, int):
                if g + 1 < ngrp:
                    idx_start(g + 1, 1 - pg)
            else:
                @pl.when(g + 1 < ngrp)
                def _():
                    idx_start(g + 1, 1 - pg)
            for u in range(gsz):
                pltpu.make_async_copy(
                    support_hbm.at[src_idx[pg].at[u]], rows[u], gsem[u]
                ).wait()
                pltpu.async_copy(
                    rows[u], agg_sh.at[dst_idx[pg].at[u]], ssem[u], add=True
                )

        idx_start(0, 0)
        idx_wait(0)
        for u in range(gsz):
            pltpu.async_copy(support_hbm.at[src_idx[0].at[u]], rows[u], gsem[u])

        pltpu.sync_copy(zeros_hbm, agg_sh.at[pl.ds(sid * ROWS_PER_TILE, ROWS_PER_TILE)])

        @pl.when(sid == 0)
        def _():
            pltpu.sync_copy(
                zeros_hbm.at[pl.ds(0, AGG_ROWS - TAIL_BASE)],
                agg_sh.at[pl.ds(TAIL_BASE, AGG_ROWS - TAIL_BASE)],
            )

        plsc.subcore_barrier()

        idx_start(1, 1)
        for u in range(gsz):
            pltpu.make_async_copy(
                support_hbm.at[src_idx[0].at[u]], rows[u], gsem[u]
            ).wait()
            pltpu.async_copy(rows[u], agg_sh.at[dst_idx[0].at[u]], ssem[u], add=True)

        def two_groups(jo, carry):
            g = 1 + 2 * jo
            group(g, 1)
            group(g + 1, 0)
            return carry

        lax.fori_loop(0, (ngrp - 1) // 2, two_groups, None)

        if ngrp % 2 == 0:
            group(ngrp - 1, (ngrp - 1) % 2)
        for u in range(gsz):
            pltpu.make_async_copy(
                rows[u], agg_sh.at[dst_idx[(ngrp - 1) % 2].at[u]], ssem[u]
            ).wait()

        plsc.subcore_barrier()
        pltpu.sync_copy(
            agg_sh.at[pl.ds(sid * ROWS_PER_TILE, ROWS_PER_TILE)],
            out_hbm.at[cid, pl.ds(sid * ROWS_PER_TILE, ROWS_PER_TILE)],
        )

        @pl.when(sid == 0)
        def _():
            pltpu.sync_copy(
                agg_sh.at[pl.ds(TAIL_BASE, TAIL)],
                out_hbm.at[cid, pl.ds(TAIL_BASE, TAIL)],
            )

    return k(support, src2d, dst2d, zeros)


def _layer1_tc(parts, w0, b0, w1, nhid, ncls):
    def body(p_ref, w0_ref, b0_ref, w1_ref, o_ref):
        agg = p_ref[0] + p_ref[1]
        h = jnp.maximum(
            jnp.dot(agg, w0_ref[...], preferred_element_type=jnp.float32) + b0_ref[...],
            0.0,
        )
        o_ref[...] = jnp.dot(h, w1_ref[...], preferred_element_type=jnp.float32)

    d = parts.shape[2]
    return pl.pallas_call(
        body,
        grid=(GRID,),
        in_specs=[
            pl.BlockSpec((NC, ROW_BLK, d), lambda i: (0, i, 0)),
            pl.BlockSpec(w0.shape, lambda i: (0, 0)),
            pl.BlockSpec((1, nhid), lambda i: (0, 0)),
            pl.BlockSpec(w1.shape, lambda i: (0, 0)),
        ],
        out_specs=pl.BlockSpec((ROW_BLK, ncls), lambda i: (i, 0)),
        out_shape=jax.ShapeDtypeStruct((N_NODES, ncls), jnp.float32),
    )(parts, w0, b0.reshape(1, nhid), w1)


def _bias_log_softmax(parts, b, n_out):
    def body(p_ref, b_ref, o_ref):
        o = p_ref[0] + p_ref[1] + b_ref[...]
        m = jnp.max(o, axis=1, keepdims=True)
        e = jnp.exp(o - m)
        s = jnp.sum(e, axis=1, keepdims=True)
        o_ref[...] = o - m - jnp.log(s)

    return pl.pallas_call(
        body,
        grid=(GRID,),
        in_specs=[
            pl.BlockSpec((NC, ROW_BLK, n_out), lambda i: (0, i, 0)),
            pl.BlockSpec((1, n_out), lambda i: (0, 0)),
        ],
        out_specs=pl.BlockSpec((ROW_BLK, n_out), lambda i: (i, 0)),
        out_shape=jax.ShapeDtypeStruct((N_NODES, n_out), jnp.float32),
    )(parts, b.reshape(1, n_out))


def kernel(x, adjs, W0, b0, W1, b1):
    src0 = adjs[0].astype(jnp.int32)
    dst0 = adjs[1].astype(jnp.int32)

    def pad_edges(n_total):
        pad = n_total - N_EDGES
        s = jnp.concatenate([src0, jnp.arange(pad, dtype=jnp.int32) * 997 % N_NODES])
        t = jnp.concatenate([dst0, N_NODES + (jnp.arange(pad, dtype=jnp.int32) % 16)])
        return s.reshape(-1, CHUNK), t.reshape(-1, CHUNK)

    nfeat = x.shape[1]
    nhid = W0.shape[1]
    ncls = W1.shape[1]
    z128 = jnp.zeros((ROWS_PER_TILE, nfeat), jnp.float32)
    z64 = jnp.zeros((ROWS_PER_TILE, ncls), jnp.float32)
    src1, dst1 = pad_edges(E_PAD1)
    src2, dst2 = pad_edges(E_PAD)

    parts1 = _seg_sum_partials_grouped(x, src1, dst1, z128, nfeat, 3, CPT1)
    s2 = _layer1_tc(parts1, W0, b0, W1, nhid, ncls)
    parts2 = _seg_sum_partials_grouped(s2, src2, dst2, z64, ncls, 8, CPT)
    return _bias_log_softmax(parts2, b1, ncls)


# speedup vs baseline: 4.2861x; 1.0191x over previous
import functools

import jax
import jax.numpy as jnp
from jax import lax
from jax.experimental import pallas as pl
from jax.experimental.pallas import tpu as pltpu
from jax.experimental.pallas import tpu_sc as plsc

N_NODES = 10000
N_EDGES = 320000
NC = 2
NS = 16
NW = NC * NS
CHUNK = 128
N_CHUNKS = N_EDGES // CHUNK
ITERS = -(-N_CHUNKS // NW)
ROWS_PER_TILE = (N_NODES // NS) // 8 * 8
TAIL_BASE = ROWS_PER_TILE * NS
TAIL = N_NODES - TAIL_BASE
CPT = 80
E_PAD = NW * CPT * CHUNK
CPT1 = 81
E_PAD1 = NW * CPT1 * CHUNK
AGG_ROWS = N_NODES + 16
NBUF = 3
NIDX = 6

ROW_BLK = 2000
GRID = N_NODES // ROW_BLK


def _seg_sum_partials(support, src, dst, zeros, d, linear_tiling=False):
    mesh = plsc.VectorSubcoreMesh(
        core_axis_name="c", subcore_axis_name="s", num_cores=NC, num_subcores=NS
    )
    params = (
        pltpu.CompilerParams(use_tc_tiling_on_sc=False) if linear_tiling else None
    )

    @functools.partial(
        pl.kernel,
        compiler_params=params,
        out_type=jax.ShapeDtypeStruct((NC, N_NODES, d), jnp.float32),
        mesh=mesh,
        scratch_types=[
            [pltpu.VMEM((CHUNK,), jnp.int32) for _ in range(NIDX)],
            [pltpu.VMEM((CHUNK,), jnp.int32) for _ in range(NIDX)],
            [pltpu.VMEM((CHUNK, d), jnp.float32) for _ in range(NBUF)],
            pltpu.VMEM_SHARED((AGG_ROWS, d), jnp.float32),
            [pltpu.SemaphoreType.DMA for _ in range(NIDX)],
            [pltpu.SemaphoreType.DMA for _ in range(NBUF)],
            [pltpu.SemaphoreType.DMA for _ in range(NBUF)],
        ],
    )
    def k(support_hbm, src_hbm, dst_hbm, zeros_hbm, out_hbm,
          src_idx, dst_idx, rows, agg_sh, isem, gsem, ssem):
        cid = lax.axis_index("c")
        sid = lax.axis_index("s")
        wid = sid * NC + cid

        pltpu.sync_copy(zeros_hbm, agg_sh.at[pl.ds(sid * ROWS_PER_TILE, ROWS_PER_TILE)])

        @pl.when(sid == 0)
        def _():
            pltpu.sync_copy(
                zeros_hbm.at[pl.ds(0, AGG_ROWS - TAIL_BASE)],
                agg_sh.at[pl.ds(TAIL_BASE, AGG_ROWS - TAIL_BASE)],
            )

        plsc.subcore_barrier()

        def gather_start(i, b):
            pltpu.async_copy(support_hbm.at[src_idx.at[i]], rows[b], gsem[b])

        def gather_wait(i, b):
            pltpu.make_async_copy(support_hbm.at[src_idx.at[i]], rows[b], gsem[b]).wait()

        def scatter_start(i, b):
            pltpu.async_copy(rows[b], agg_sh.at[dst_idx.at[i]], ssem[b], add=True)

        def scatter_wait(i, b):
            pltpu.make_async_copy(rows[b], agg_sh.at[dst_idx.at[i]], ssem[b]).wait()

        def idx_start(i, bi):
            base = (wid * CPT + i) * CHUNK
            pltpu.async_copy(src_hbm.at[pl.ds(base, CHUNK)], src_idx[bi], isem[bi])
            pltpu.async_copy(dst_hbm.at[pl.ds(base, CHUNK)], dst_idx[bi], isem[bi])

        def idx_wait(bi):
            pltpu.make_async_copy(src_hbm.at[pl.ds(0, CHUNK)], src_idx[bi], isem[bi]).wait()
            pltpu.make_async_copy(dst_hbm.at[pl.ds(0, CHUNK)], dst_idx[bi], isem[bi]).wait()

        def gather_start(br, bi):
            pltpu.async_copy(support_hbm.at[src_idx[bi]], rows[br], gsem[br])

        def gather_wait(br, bi):
            pltpu.make_async_copy(support_hbm.at[src_idx[bi]], rows[br], gsem[br]).wait()

        def scatter_start(br, bi):
            pltpu.async_copy(rows[br], agg_sh.at[dst_idx[bi]], ssem[br], add=True)

        def scatter_wait(br, bi):
            pltpu.make_async_copy(rows[br], agg_sh.at[dst_idx[bi]], ssem[br]).wait()

        def step(i, ph, with_g=True, with_i=True, with_sw=True):
            br, bi = ph % NBUF, ph % NIDX
            gather_wait(br, bi)
            scatter_start(br, bi)
            if with_g:
                brn, bin_ = (ph + 1) % NBUF, (ph + 1) % NIDX
                idx_wait(bin_)
                if with_sw:
                    scatter_wait(brn, (ph + 4) % NIDX)
                gather_start(brn, bin_)
            if with_i:
                idx_start(i + 3, (ph + 3) % NIDX)

        for j in range(NBUF):
            idx_start(j, j)
        idx_wait(0)
        gather_start(0, 0)
        step(0, 0, with_sw=False)
        step(1, 1, with_sw=False)

        def group(io, carry):
            i0 = 2 + io * NIDX
            for u in range(NIDX):
                step(i0 + u, (2 + u) % NIDX)
            return carry

        lax.fori_loop(0, (CPT - 8) // NIDX, group, None)

        step(74, 2)
        step(75, 3)
        step(76, 4)
        step(77, 5, with_i=False)
        step(78, 0, with_i=False)
        step(79, 1, with_g=False, with_i=False)
        scatter_wait(2, 5)
        scatter_wait(0, 0)
        scatter_wait(1, 1)

        plsc.subcore_barrier()
        pltpu.sync_copy(
            agg_sh.at[pl.ds(sid * ROWS_PER_TILE, ROWS_PER_TILE)],
            out_hbm.at[cid, pl.ds(sid * ROWS_PER_TILE, ROWS_PER_TILE)],
        )

        @pl.when(sid == 0)
        def _():
            pltpu.sync_copy(
                agg_sh.at[pl.ds(TAIL_BASE, TAIL)],
                out_hbm.at[cid, pl.ds(TAIL_BASE, TAIL)],
            )

    return k(support, src, dst, zeros)


def _seg_sum_partials_grouped(support, src2d, dst2d, zeros, d, gsz, cpt):
    ngrp = cpt // gsz
    mesh = plsc.VectorSubcoreMesh(
        core_axis_name="c", subcore_axis_name="s", num_cores=NC, num_subcores=NS
    )

    @functools.partial(
        pl.kernel,
        compiler_params=pltpu.CompilerParams(use_tc_tiling_on_sc=False),
        out_type=jax.ShapeDtypeStruct((NC, N_NODES, d), jnp.float32),
        mesh=mesh,
        scratch_types=[
            [pltpu.VMEM((gsz, CHUNK), jnp.int32) for _ in range(2)],
            [pltpu.VMEM((gsz, CHUNK), jnp.int32) for _ in range(2)],
            [pltpu.VMEM((CHUNK, d), jnp.float32) for _ in range(gsz)],
            pltpu.VMEM_SHARED((AGG_ROWS, d), jnp.float32),
            [pltpu.SemaphoreType.DMA for _ in range(2)],
            [pltpu.SemaphoreType.DMA for _ in range(gsz)],
            [pltpu.SemaphoreType.DMA for _ in range(gsz)],
        ],
    )
    def k(support_hbm, src_hbm, dst_hbm, zeros_hbm, out_hbm,
          src_idx, dst_idx, rows, agg_sh, isem, gsem, ssem):
        cid = lax.axis_index("c")
        sid = lax.axis_index("s")
        wid = sid * NC + cid

        def idx_start(g, pg):
            row0 = wid * cpt + g * gsz
            pltpu.async_copy(src_hbm.at[pl.ds(row0, gsz)], src_idx[pg], isem[pg])
            pltpu.async_copy(dst_hbm.at[pl.ds(row0, gsz)], dst_idx[pg], isem[pg])

        def idx_wait(pg):
            pltpu.make_async_copy(src_hbm.at[pl.ds(0, gsz)], src_idx[pg], isem[pg]).wait()
            pltpu.make_async_copy(dst_hbm.at[pl.ds(0, gsz)], dst_idx[pg], isem[pg]).wait()

        def group(g, pg, first=False):
            idx_wait(pg)
            for u in range(gsz):
                if not first:
                    pltpu.make_async_copy(
                        rows[u], agg_sh.at[dst_idx[1 - pg].at[u]], ssem[u]
                    ).wait()
                pltpu.async_copy(support_hbm.at[src_idx[pg].at[u]], rows[u], gsem[u])
            if isinstance(g, int):
                if g + 1 < ngrp:
                    idx_start(g + 1, 1 - pg)
            else:
                @pl.when(g + 1 < ngrp)
                def _():
                    idx_start(g + 1, 1 - pg)
            for u in range(gsz):
                pltpu.make_async_copy(
                    support_hbm.at[src_idx[pg].at[u]], rows[u], gsem[u]
                ).wait()
                pltpu.async_copy(
                    rows[u], agg_sh.at[dst_idx[pg].at[u]], ssem[u], add=True
                )

        idx_start(0, 0)
        idx_wait(0)
        for u in range(gsz):
            pltpu.async_copy(support_hbm.at[src_idx[0].at[u]], rows[u], gsem[u])

        pltpu.sync_copy(zeros_hbm, agg_sh.at[pl.ds(sid * ROWS_PER_TILE, ROWS_PER_TILE)])

        @pl.when(sid == 0)
        def _():
            pltpu.sync_copy(
                zeros_hbm.at[pl.ds(0, AGG_ROWS - TAIL_BASE)],
                agg_sh.at[pl.ds(TAIL_BASE, AGG_ROWS - TAIL_BASE)],
            )

        plsc.subcore_barrier()

        idx_start(1, 1)
        for u in range(gsz):
            pltpu.make_async_copy(
                support_hbm.at[src_idx[0].at[u]], rows[u], gsem[u]
            ).wait()
            pltpu.async_copy(rows[u], agg_sh.at[dst_idx[0].at[u]], ssem[u], add=True)

        def two_groups(jo, carry):
            g = 1 + 2 * jo
            group(g, 1)
            group(g + 1, 0)
            return carry

        lax.fori_loop(0, (ngrp - 1) // 2, two_groups, None)

        if ngrp % 2 == 0:
            group(ngrp - 1, (ngrp - 1) % 2)
        for u in range(gsz):
            pltpu.make_async_copy(
                rows[u], agg_sh.at[dst_idx[(ngrp - 1) % 2].at[u]], ssem[u]
            ).wait()

        plsc.subcore_barrier()
        pltpu.sync_copy(
            agg_sh.at[pl.ds(sid * ROWS_PER_TILE, ROWS_PER_TILE)],
            out_hbm.at[cid, pl.ds(sid * ROWS_PER_TILE, ROWS_PER_TILE)],
        )

        @pl.when(sid == 0)
        def _():
            pltpu.sync_copy(
                agg_sh.at[pl.ds(TAIL_BASE, TAIL)],
                out_hbm.at[cid, pl.ds(TAIL_BASE, TAIL)],
            )

    return k(support, src2d, dst2d, zeros)


def _layer1_tc(parts, w0, b0, w1, nhid, ncls):
    def body(p_ref, w0_ref, b0_ref, w1_ref, o_ref):
        agg = p_ref[0] + p_ref[1]
        h = jnp.maximum(
            jnp.dot(agg, w0_ref[...], preferred_element_type=jnp.float32) + b0_ref[...],
            0.0,
        )
        o_ref[...] = jnp.dot(h, w1_ref[...], preferred_element_type=jnp.float32)

    d = parts.shape[2]
    return pl.pallas_call(
        body,
        grid=(GRID,),
        in_specs=[
            pl.BlockSpec((NC, ROW_BLK, d), lambda i: (0, i, 0)),
            pl.BlockSpec(w0.shape, lambda i: (0, 0)),
            pl.BlockSpec((1, nhid), lambda i: (0, 0)),
            pl.BlockSpec(w1.shape, lambda i: (0, 0)),
        ],
        out_specs=pl.BlockSpec((ROW_BLK, ncls), lambda i: (i, 0)),
        out_shape=jax.ShapeDtypeStruct((N_NODES, ncls), jnp.float32),
    )(parts, w0, b0.reshape(1, nhid), w1)


def _bias_log_softmax(parts, b, n_out):
    def body(p_ref, b_ref, o_ref):
        o = p_ref[0] + p_ref[1] + b_ref[...]
        m = jnp.max(o, axis=1, keepdims=True)
        e = jnp.exp(o - m)
        s = jnp.sum(e, axis=1, keepdims=True)
        o_ref[...] = o - m - jnp.log(s)

    return pl.pallas_call(
        body,
        grid=(GRID,),
        in_specs=[
            pl.BlockSpec((NC, ROW_BLK, n_out), lambda i: (0, i, 0)),
            pl.BlockSpec((1, n_out), lambda i: (0, 0)),
        ],
        out_specs=pl.BlockSpec((ROW_BLK, n_out), lambda i: (i, 0)),
        out_shape=jax.ShapeDtypeStruct((N_NODES, n_out), jnp.float32),
    )(parts, b.reshape(1, n_out))


def kernel(x, adjs, W0, b0, W1, b1):
    src0 = adjs[0].astype(jnp.int32)
    dst0 = adjs[1].astype(jnp.int32)

    def pad_edges(n_total):
        pad = n_total - N_EDGES
        s = jnp.concatenate([src0, jnp.arange(pad, dtype=jnp.int32) * 997 % N_NODES])
        t = jnp.concatenate([dst0, N_NODES + (jnp.arange(pad, dtype=jnp.int32) % 16)])
        return s.reshape(-1, CHUNK), t.reshape(-1, CHUNK)

    nfeat = x.shape[1]
    nhid = W0.shape[1]
    ncls = W1.shape[1]
    z128 = jnp.zeros((ROWS_PER_TILE, nfeat), jnp.float32)
    z64 = jnp.zeros((ROWS_PER_TILE, ncls), jnp.float32)
    src1, dst1 = pad_edges(E_PAD1)
    src2, dst2 = pad_edges(E_PAD)

    parts1 = _seg_sum_partials_grouped(x, src1, dst1, z128, nfeat, 3, CPT1)
    s2 = _layer1_tc(parts1, W0, b0, W1, nhid, ncls)
    parts2 = _seg_sum_partials_grouped(s2, src2, dst2, z64, ncls, 8, CPT)
    return _bias_log_softmax(parts2, b1, ncls)


# speedup vs baseline: 4.3465x; 1.0141x over previous
import functools

import jax
import jax.numpy as jnp
from jax import lax
from jax.experimental import pallas as pl
from jax.experimental.pallas import tpu as pltpu
from jax.experimental.pallas import tpu_sc as plsc

N_NODES = 10000
N_EDGES = 320000
NC = 2
NS = 16
NW = NC * NS
CHUNK = 128
ROWS_PER_TILE = (N_NODES // NS) // 8 * 8
TAIL_BASE = ROWS_PER_TILE * NS
TAIL = N_NODES - TAIL_BASE
CPT = 80
E_PAD = NW * CPT * CHUNK
CPT1 = 81
E_PAD1 = NW * CPT1 * CHUNK
AGG_ROWS = N_NODES + 16

ROW_BLK = 5000
GRID = N_NODES // ROW_BLK


def _seg_sum_partials_grouped(support, src2d, dst2d, zeros, d, gsz, cpt):
    ngrp = cpt // gsz
    mesh = plsc.VectorSubcoreMesh(
        core_axis_name="c", subcore_axis_name="s", num_cores=NC, num_subcores=NS
    )

    @functools.partial(
        pl.kernel,
        compiler_params=pltpu.CompilerParams(use_tc_tiling_on_sc=False),
        out_type=jax.ShapeDtypeStruct((NC, N_NODES, d), jnp.float32),
        mesh=mesh,
        scratch_types=[
            [pltpu.VMEM((gsz, CHUNK), jnp.int32) for _ in range(2)],
            [pltpu.VMEM((gsz, CHUNK), jnp.int32) for _ in range(2)],
            [pltpu.VMEM((CHUNK, d), jnp.float32) for _ in range(gsz)],
            pltpu.VMEM_SHARED((AGG_ROWS, d), jnp.float32),
            [pltpu.SemaphoreType.DMA for _ in range(2)],
            [pltpu.SemaphoreType.DMA for _ in range(gsz)],
            [pltpu.SemaphoreType.DMA for _ in range(gsz)],
        ],
    )
    def k(support_hbm, src_hbm, dst_hbm, zeros_hbm, out_hbm,
          src_idx, dst_idx, rows, agg_sh, isem, gsem, ssem):
        cid = lax.axis_index("c")
        sid = lax.axis_index("s")
        wid = sid * NC + cid

        def idx_start(g, pg):
            row0 = wid * cpt + g * gsz
            pltpu.async_copy(src_hbm.at[pl.ds(row0, gsz)], src_idx[pg], isem[pg])
            pltpu.async_copy(dst_hbm.at[pl.ds(row0, gsz)], dst_idx[pg], isem[pg])

        def idx_wait(pg):
            pltpu.make_async_copy(src_hbm.at[pl.ds(0, gsz)], src_idx[pg], isem[pg]).wait()
            pltpu.make_async_copy(dst_hbm.at[pl.ds(0, gsz)], dst_idx[pg], isem[pg]).wait()

        def group(g, pg, first=False):
            idx_wait(pg)
            for u in range(gsz):
                if not first:
                    pltpu.make_async_copy(
                        rows[u], agg_sh.at[dst_idx[1 - pg].at[u]], ssem[u]
                    ).wait()
                pltpu.async_copy(support_hbm.at[src_idx[pg].at[u]], rows[u], gsem[u])
            if isinstance(g, int):
                if g + 1 < ngrp:
                    idx_start(g + 1, 1 - pg)
            else:
                @pl.when(g + 1 < ngrp)
                def _():
                    idx_start(g + 1, 1 - pg)
            for u in range(gsz):
                pltpu.make_async_copy(
                    support_hbm.at[src_idx[pg].at[u]], rows[u], gsem[u]
                ).wait()
                pltpu.async_copy(
                    rows[u], agg_sh.at[dst_idx[pg].at[u]], ssem[u], add=True
                )

        idx_start(0, 0)
        idx_wait(0)
        for u in range(gsz):
            pltpu.async_copy(support_hbm.at[src_idx[0].at[u]], rows[u], gsem[u])

        pltpu.sync_copy(zeros_hbm, agg_sh.at[pl.ds(sid * ROWS_PER_TILE, ROWS_PER_TILE)])

        @pl.when(sid == 0)
        def _():
            pltpu.sync_copy(
                zeros_hbm.at[pl.ds(0, AGG_ROWS - TAIL_BASE)],
                agg_sh.at[pl.ds(TAIL_BASE, AGG_ROWS - TAIL_BASE)],
            )

        plsc.subcore_barrier()

        idx_start(1, 1)
        for u in range(gsz):
            pltpu.make_async_copy(
                support_hbm.at[src_idx[0].at[u]], rows[u], gsem[u]
            ).wait()
            pltpu.async_copy(rows[u], agg_sh.at[dst_idx[0].at[u]], ssem[u], add=True)

        def two_groups(jo, carry):
            g = 1 + 2 * jo
            group(g, 1)
            group(g + 1, 0)
            return carry

        lax.fori_loop(0, (ngrp - 1) // 2, two_groups, None)

        if ngrp % 2 == 0:
            group(ngrp - 1, (ngrp - 1) % 2)
        for u in range(gsz):
            pltpu.make_async_copy(
                rows[u], agg_sh.at[dst_idx[(ngrp - 1) % 2].at[u]], ssem[u]
            ).wait()

        plsc.subcore_barrier()
        pltpu.sync_copy(
            agg_sh.at[pl.ds(sid * ROWS_PER_TILE, ROWS_PER_TILE)],
            out_hbm.at[cid, pl.ds(sid * ROWS_PER_TILE, ROWS_PER_TILE)],
        )

        @pl.when(sid == 0)
        def _():
            pltpu.sync_copy(
                agg_sh.at[pl.ds(TAIL_BASE, TAIL)],
                out_hbm.at[cid, pl.ds(TAIL_BASE, TAIL)],
            )

    return k(support, src2d, dst2d, zeros)


def _layer1_tc(parts, w0, b0, w1, nhid, ncls):
    def body(p_ref, w0_ref, b0_ref, w1_ref, o_ref):
        agg = p_ref[0] + p_ref[1]
        h = jnp.maximum(
            jnp.dot(agg, w0_ref[...], preferred_element_type=jnp.float32) + b0_ref[...],
            0.0,
        )
        o_ref[...] = jnp.dot(h, w1_ref[...], preferred_element_type=jnp.float32)

    d = parts.shape[2]
    return pl.pallas_call(
        body,
        grid=(GRID,),
        in_specs=[
            pl.BlockSpec((NC, ROW_BLK, d), lambda i: (0, i, 0)),
            pl.BlockSpec(w0.shape, lambda i: (0, 0)),
            pl.BlockSpec((1, nhid), lambda i: (0, 0)),
            pl.BlockSpec(w1.shape, lambda i: (0, 0)),
        ],
        out_specs=pl.BlockSpec((ROW_BLK, ncls), lambda i: (i, 0)),
        out_shape=jax.ShapeDtypeStruct((N_NODES, ncls), jnp.float32),
    )(parts, w0, b0.reshape(1, nhid), w1)


def _bias_log_softmax(parts, b, n_out):
    def body(p_ref, b_ref, o_ref):
        o = p_ref[0] + p_ref[1] + b_ref[...]
        m = jnp.max(o, axis=1, keepdims=True)
        e = jnp.exp(o - m)
        s = jnp.sum(e, axis=1, keepdims=True)
        o_ref[...] = o - m - jnp.log(s)

    return pl.pallas_call(
        body,
        grid=(GRID,),
        in_specs=[
            pl.BlockSpec((NC, ROW_BLK, n_out), lambda i: (0, i, 0)),
            pl.BlockSpec((1, n_out), lambda i: (0, 0)),
        ],
        out_specs=pl.BlockSpec((ROW_BLK, n_out), lambda i: (i, 0)),
        out_shape=jax.ShapeDtypeStruct((N_NODES, n_out), jnp.float32),
    )(parts, b.reshape(1, n_out))


def kernel(x, adjs, W0, b0, W1, b1):
    src0 = adjs[0].astype(jnp.int32)
    dst0 = adjs[1].astype(jnp.int32)

    def pad_edges(n_total):
        pad = n_total - N_EDGES
        s = jnp.concatenate([src0, jnp.arange(pad, dtype=jnp.int32) * 997 % N_NODES])
        t = jnp.concatenate([dst0, N_NODES + (jnp.arange(pad, dtype=jnp.int32) % 16)])
        return s.reshape(-1, CHUNK), t.reshape(-1, CHUNK)

    nfeat = x.shape[1]
    nhid = W0.shape[1]
    ncls = W1.shape[1]
    z128 = jnp.zeros((ROWS_PER_TILE, nfeat), jnp.float32)
    z64 = jnp.zeros((ROWS_PER_TILE, ncls), jnp.float32)
    src1, dst1 = pad_edges(E_PAD1)
    src2, dst2 = pad_edges(E_PAD)

    parts1 = _seg_sum_partials_grouped(x, src1, dst1, z128, nfeat, 3, CPT1)
    s2 = _layer1_tc(parts1, W0, b0, W1, nhid, ncls)
    parts2 = _seg_sum_partials_grouped(s2, src2, dst2, z64, ncls, 8, CPT)
    return _bias_log_softmax(parts2, b1, ncls)
